# baseline (speedup 1.0000x reference)
import functools

import jax
import jax.numpy as jnp
from jax import lax
from jax.experimental import pallas as pl
from jax.experimental.pallas import tpu as pltpu
from jax.experimental.pallas import tpu_sc as plsc

N = 10000
NPAD = 10240
E = 320000
NC, NS = 2, 16
NW = NC * NS
C = 64
EPAD = 327680
NCH = EPAD // (NW * C)
DRAIN = NPAD // NS
BLK = 512
GRID = NPAD // BLK
DEGW = 128

_mesh = plsc.VectorSubcoreMesh(core_axis_name="c", subcore_axis_name="s")


@functools.partial(
    pl.kernel,
    out_type=jax.ShapeDtypeStruct((NC, NPAD, DEGW), jnp.float32),
    mesh=_mesh,
    scratch_types=[
        pltpu.VMEM((4, 2, C), jnp.int32),
        pltpu.VMEM((C, DEGW), jnp.float32),
        pltpu.VMEM_SHARED((NPAD, DEGW), jnp.float32),
        pltpu.SemaphoreType.DMA,
        pltpu.SemaphoreType.DMA,
        pltpu.SemaphoreType.DMA,
        pltpu.SemaphoreType.DMA,
        pltpu.SemaphoreType.DMA,
        pltpu.SemaphoreType.DMA,
        pltpu.SemaphoreType.DMA,
        pltpu.SemaphoreType.DMA,
    ],
)
def _sc_deg(idx_hbm, zeros_hbm, ones_hbm, out_hbm, idxv, onesv, acc,
            i0, i1, i2, i3, s0, s1, s2, s3):
    cid = lax.axis_index("c")
    sid = lax.axis_index("s")
    wid = cid * NS + sid
    base = wid * NCH
    isems = [i0, i1, i2, i3]
    ssems = [s0, s1, s2, s3]

    def issue_idx(j, ib):
        pltpu.async_copy(idx_hbm.at[j + base], idxv.at[ib], isems[ib])

    def wait_idx(j, ib):
        pltpu.make_async_copy(idx_hbm.at[j + base], idxv.at[ib],
                              isems[ib]).wait()

    def issue_scatter(ib):
        pltpu.async_copy(onesv, acc.at[idxv.at[ib].at[1]], ssems[ib],
                         add=True)

    def wait_scatter(ib):
        pltpu.make_async_copy(onesv, acc.at[idxv.at[ib].at[1]],
                              ssems[ib]).wait()

    issue_idx(0, 0)
    issue_idx(1, 1)
    issue_idx(2, 2)
    pltpu.sync_copy(ones_hbm, onesv)
    pltpu.sync_copy(zeros_hbm, acc.at[pl.ds(sid * DRAIN, DRAIN)])
    plsc.subcore_barrier()

    @pl.loop(0, NCH, step=4)
    def _chunk(o):
        for b4 in range(4):
            j = o + b4
            ib = b4 % 4
            fib = (b4 + 3) % 4

            @pl.when(j + 3 < NCH)
            def _prefetch():
                @pl.when(j >= 1)
                def _drain_prev():
                    wait_scatter(fib)

                issue_idx(j + 3, fib)

            wait_idx(j, ib)
            issue_scatter(ib)

    wait_scatter(0)
    wait_scatter(1)
    wait_scatter(2)
    wait_scatter(3)
    plsc.subcore_barrier()
    pltpu.sync_copy(acc.at[pl.ds(sid * DRAIN, DRAIN)],
                    out_hbm.at[cid].at[pl.ds(sid * DRAIN, DRAIN)])


@functools.partial(
    pl.kernel,
    out_type=jax.ShapeDtypeStruct((NC, NPAD, 128), jnp.float32),
    mesh=_mesh,
    scratch_types=[
        pltpu.VMEM((8, 2, C), jnp.int32),
        pltpu.VMEM((4, C, 128), jnp.float32),
        pltpu.VMEM_SHARED((NPAD, 128), jnp.float32),
    ] + [pltpu.SemaphoreType.DMA] * 16,
)
def _sc_apply(u_hbm, idx_hbm, zeros_hbm, out_hbm, idxv, rows, acc, *sems):
    cid = lax.axis_index("c")
    sid = lax.axis_index("s")
    wid = cid * NS + sid
    base = wid * NCH
    isems = list(sems[0:8])
    gs = list(sems[8:12])
    ss = list(sems[12:16])

    def issue_idx(j, ib):
        pltpu.async_copy(idx_hbm.at[j + base], idxv.at[ib], isems[ib])

    def wait_idx(j, ib):
        pltpu.make_async_copy(idx_hbm.at[j + base], idxv.at[ib],
                              isems[ib]).wait()

    def issue_gather(ib, rb):
        pltpu.async_copy(u_hbm.at[idxv.at[ib].at[0]], rows.at[rb], gs[rb])

    def wait_gather(ib, rb):
        pltpu.make_async_copy(u_hbm.at[idxv.at[ib].at[0]], rows.at[rb],
                              gs[rb]).wait()

    def issue_scatter(ib, rb):
        pltpu.async_copy(rows.at[rb], acc.at[idxv.at[ib].at[1]], ss[rb],
                         add=True)

    def wait_scatter(ib, rb):
        pltpu.make_async_copy(rows.at[rb], acc.at[idxv.at[ib].at[1]],
                              ss[rb]).wait()

    for k in range(5):
        issue_idx(k, k)
    pltpu.sync_copy(zeros_hbm, acc.at[pl.ds(sid * DRAIN, DRAIN)])
    wait_idx(0, 0)
    issue_gather(0, 0)
    plsc.subcore_barrier()

    @pl.loop(0, NCH, step=8)
    def _pipe(o):
        for b8 in range(8):
            j = o + b8
            ib = b8 % 8
            rb = b8 % 4
            nib = (b8 + 1) % 8
            nrb = (b8 + 1) % 4
            fib = (b8 + 5) % 8
            frb = (b8 + 1) % 4

            @pl.when(j + 1 < NCH)
            def _advance():
                @pl.when(j >= 3)
                def _drain_prev():
                    wait_scatter(fib, frb)

                @pl.when(j + 5 < NCH)
                def _prefetch_idx():
                    issue_idx(j + 5, fib)

                wait_idx(j + 1, nib)
                issue_gather(nib, nrb)

            wait_gather(ib, rb)
            issue_scatter(ib, rb)

    wait_scatter((NCH - 4) % 8, 0)
    wait_scatter((NCH - 3) % 8, 1)
    wait_scatter((NCH - 2) % 8, 2)
    wait_scatter((NCH - 1) % 8, 3)
    plsc.subcore_barrier()
    pltpu.sync_copy(acc.at[pl.ds(sid * DRAIN, DRAIN)],
                    out_hbm.at[cid].at[pl.ds(sid * DRAIN, DRAIN)])


def _k2_body(x_ref, degp_ref, u1_ref, dis_ref):
    j = pl.program_id(0)
    p = degp_ref[...]
    deg = p[0, :, 0:1] + p[1, :, 0:1] + 1.0
    dis = lax.rsqrt(deg)
    rows = lax.broadcasted_iota(jnp.int32, (BLK, 1), 0) + j * BLK
    mask = rows < N
    u1_ref[...] = jnp.where(mask, x_ref[...] * dis, 0.0)
    dis_ref[...] = dis


_tc_k2 = pl.pallas_call(
    _k2_body,
    grid=(GRID,),
    in_specs=[
        pl.BlockSpec((BLK, 128), lambda j: (j, 0)),
        pl.BlockSpec((NC, BLK, DEGW), lambda j: (0, j, 0)),
    ],
    out_specs=[
        pl.BlockSpec((BLK, 128), lambda j: (j, 0)),
        pl.BlockSpec((BLK, 1), lambda j: (j, 0)),
    ],
    out_shape=[
        jax.ShapeDtypeStruct((NPAD, 128), jnp.float32),
        jax.ShapeDtypeStruct((NPAD, 1), jnp.float32),
    ],
)


def _k4_body(s_ref, u1_ref, dis_ref, w_ref, b_ref, u2_ref):
    j = pl.program_id(0)
    s = s_ref[0] + s_ref[1] + u1_ref[...]
    ax = dis_ref[...] * s
    h = jnp.dot(ax, w_ref[...], preferred_element_type=jnp.float32)
    h = jnp.maximum(h + b_ref[...], 0.0)
    rows = lax.broadcasted_iota(jnp.int32, (BLK, 1), 0) + j * BLK
    mask = rows < N
    u2_ref[...] = jnp.where(mask, dis_ref[...] * h, 0.0)


_tc_k4 = pl.pallas_call(
    _k4_body,
    grid=(GRID,),
    in_specs=[
        pl.BlockSpec((NC, BLK, 128), lambda j: (0, j, 0)),
        pl.BlockSpec((BLK, 128), lambda j: (j, 0)),
        pl.BlockSpec((BLK, 1), lambda j: (j, 0)),
        pl.BlockSpec((128, 128), lambda j: (0, 0)),
        pl.BlockSpec((1, 128), lambda j: (0, 0)),
    ],
    out_specs=pl.BlockSpec((BLK, 128), lambda j: (j, 0)),
    out_shape=jax.ShapeDtypeStruct((NPAD, 128), jnp.float32),
)


def _k6_body(s_ref, u2_ref, dis_ref, wmu_ref, bmu_ref, wls_ref, bls_ref,
             mu_ref, ls_ref):
    s = s_ref[0] + s_ref[1] + u2_ref[...]
    ah = dis_ref[...] * s
    mu_ref[...] = (
        jnp.dot(ah, wmu_ref[...], preferred_element_type=jnp.float32)
        + bmu_ref[...])
    ls_ref[...] = (
        jnp.dot(ah, wls_ref[...], preferred_element_type=jnp.float32)
        + bls_ref[...])


_tc_k6 = pl.pallas_call(
    _k6_body,
    grid=(GRID,),
    in_specs=[
        pl.BlockSpec((NC, BLK, 128), lambda j: (0, j, 0)),
        pl.BlockSpec((BLK, 128), lambda j: (j, 0)),
        pl.BlockSpec((BLK, 1), lambda j: (j, 0)),
        pl.BlockSpec((128, 128), lambda j: (0, 0)),
        pl.BlockSpec((1, 128), lambda j: (0, 0)),
        pl.BlockSpec((128, 128), lambda j: (0, 0)),
        pl.BlockSpec((1, 128), lambda j: (0, 0)),
    ],
    out_specs=[
        pl.BlockSpec((BLK, 128), lambda j: (j, 0)),
        pl.BlockSpec((BLK, 128), lambda j: (j, 0)),
    ],
    out_shape=[
        jax.ShapeDtypeStruct((N, 128), jnp.float32),
        jax.ShapeDtypeStruct((N, 128), jnp.float32),
    ],
)


def kernel(x, edge_index, W1, b1, Wmu, bmu, Wls, bls):
    ei = edge_index.astype(jnp.int32)
    npads = EPAD - E
    pad_idx = N + (jnp.arange(npads, dtype=jnp.int32) % (NPAD - N))
    srcp = jnp.concatenate([ei[0], pad_idx]).reshape(EPAD // C, 1, C)
    dstp = jnp.concatenate([ei[1], pad_idx]).reshape(EPAD // C, 1, C)
    idx = jnp.concatenate([srcp, dstp], axis=1)

    ones = jnp.ones((C, DEGW), jnp.float32)
    zeros128 = jnp.zeros((DRAIN, 128), jnp.float32)

    degp = _sc_deg(idx, zeros128, ones)
    u1, dis = _tc_k2(x, degp)
    s1 = _sc_apply(u1, idx, zeros128)
    u2 = _tc_k4(s1, u1, dis, W1, b1.reshape(1, 128))
    s2 = _sc_apply(u2, idx, zeros128)
    mu, ls = _tc_k6(s2, u2, dis, Wmu, bmu.reshape(1, 128),
                    Wls, bls.reshape(1, 128))
    return (mu, ls)


# speedup vs baseline: 1.0920x; 1.0920x over previous
import functools

import jax
import jax.numpy as jnp
from jax import lax
from jax.experimental import pallas as pl
from jax.experimental.pallas import tpu as pltpu
from jax.experimental.pallas import tpu_sc as plsc

N = 10000
NPAD = 10240
E = 320000
NC, NS = 2, 16
NW = NC * NS
C = 128
EPAD = 327680
NCH = EPAD // (NW * C)
DRAIN = NPAD // NS
BLK = 512
GRID = NPAD // BLK
DEGW = 128

_mesh = plsc.VectorSubcoreMesh(core_axis_name="c", subcore_axis_name="s")


@functools.partial(
    pl.kernel,
    out_type=jax.ShapeDtypeStruct((NC, NPAD, DEGW), jnp.float32),
    mesh=_mesh,
    scratch_types=[
        pltpu.VMEM((4, 2, C), jnp.int32),
        pltpu.VMEM((C, DEGW), jnp.float32),
        pltpu.VMEM_SHARED((NPAD, DEGW), jnp.float32),
        pltpu.SemaphoreType.DMA,
        pltpu.SemaphoreType.DMA,
        pltpu.SemaphoreType.DMA,
        pltpu.SemaphoreType.DMA,
        pltpu.SemaphoreType.DMA,
        pltpu.SemaphoreType.DMA,
        pltpu.SemaphoreType.DMA,
        pltpu.SemaphoreType.DMA,
    ],
)
def _sc_deg(idx_hbm, zeros_hbm, ones_hbm, out_hbm, idxv, onesv, acc,
            i0, i1, i2, i3, s0, s1, s2, s3):
    cid = lax.axis_index("c")
    sid = lax.axis_index("s")
    wid = cid * NS + sid
    base = wid * NCH
    isems = [i0, i1, i2, i3]
    ssems = [s0, s1, s2, s3]

    def issue_idx(j, ib):
        pltpu.async_copy(idx_hbm.at[j + base], idxv.at[ib], isems[ib])

    def wait_idx(j, ib):
        pltpu.make_async_copy(idx_hbm.at[j + base], idxv.at[ib],
                              isems[ib]).wait()

    def issue_scatter(ib):
        pltpu.async_copy(onesv, acc.at[idxv.at[ib].at[1]], ssems[ib],
                         add=True)

    def wait_scatter(ib):
        pltpu.make_async_copy(onesv, acc.at[idxv.at[ib].at[1]],
                              ssems[ib]).wait()

    issue_idx(0, 0)
    issue_idx(1, 1)
    issue_idx(2, 2)
    pltpu.sync_copy(ones_hbm, onesv)
    pltpu.sync_copy(zeros_hbm, acc.at[pl.ds(sid * DRAIN, DRAIN)])
    plsc.subcore_barrier()

    @pl.loop(0, NCH, step=4)
    def _chunk(o):
        for b4 in range(4):
            j = o + b4
            ib = b4 % 4
            fib = (b4 + 3) % 4

            @pl.when(j + 3 < NCH)
            def _prefetch():
                @pl.when(j >= 1)
                def _drain_prev():
                    wait_scatter(fib)

                issue_idx(j + 3, fib)

            wait_idx(j, ib)
            issue_scatter(ib)

    wait_scatter(0)
    wait_scatter(1)
    wait_scatter(2)
    wait_scatter(3)
    plsc.subcore_barrier()
    pltpu.sync_copy(acc.at[pl.ds(sid * DRAIN, DRAIN)],
                    out_hbm.at[cid].at[pl.ds(sid * DRAIN, DRAIN)])


@functools.partial(
    pl.kernel,
    out_type=jax.ShapeDtypeStruct((NC, NPAD, 128), jnp.float32),
    mesh=_mesh,
    scratch_types=[
        pltpu.VMEM((4, 2, C), jnp.int32),
        pltpu.VMEM((2, C, 128), jnp.float32),
        pltpu.VMEM_SHARED((NPAD, 128), jnp.float32),
        pltpu.SemaphoreType.DMA,
        pltpu.SemaphoreType.DMA,
        pltpu.SemaphoreType.DMA,
        pltpu.SemaphoreType.DMA,
        pltpu.SemaphoreType.DMA,
        pltpu.SemaphoreType.DMA,
        pltpu.SemaphoreType.DMA,
        pltpu.SemaphoreType.DMA,
    ],
)
def _sc_apply(u_hbm, idx_hbm, zeros_hbm, out_hbm,
              idxv, rows, acc, i0, i1, i2, i3, g0, g1, s0, s1):
    cid = lax.axis_index("c")
    sid = lax.axis_index("s")
    wid = cid * NS + sid
    base = wid * NCH
    isems = [i0, i1, i2, i3]
    gs = [g0, g1]
    ss = [s0, s1]

    def issue_idx(j, ib):
        pltpu.async_copy(idx_hbm.at[j + base], idxv.at[ib], isems[ib])

    def wait_idx(j, ib):
        pltpu.make_async_copy(idx_hbm.at[j + base], idxv.at[ib],
                              isems[ib]).wait()

    def issue_gather(j, ib, rb):
        pltpu.async_copy(u_hbm.at[idxv.at[ib].at[0]], rows.at[rb], gs[rb])

    def wait_gather(ib, rb):
        pltpu.make_async_copy(u_hbm.at[idxv.at[ib].at[0]], rows.at[rb],
                              gs[rb]).wait()

    def issue_scatter(ib, rb):
        pltpu.async_copy(rows.at[rb], acc.at[idxv.at[ib].at[1]], ss[rb],
                         add=True)

    def wait_scatter(ib, rb):
        pltpu.make_async_copy(rows.at[rb], acc.at[idxv.at[ib].at[1]],
                              ss[rb]).wait()

    issue_idx(0, 0)
    issue_idx(1, 1)
    issue_idx(2, 2)
    pltpu.sync_copy(zeros_hbm, acc.at[pl.ds(sid * DRAIN, DRAIN)])
    wait_idx(0, 0)
    issue_gather(0, 0, 0)
    plsc.subcore_barrier()

    @pl.loop(0, NCH, step=4)
    def _pipe(o):
        for b4 in range(4):
            j = o + b4
            rb = b4 % 2
            nrb = 1 - rb
            ib = b4 % 4
            nib = (b4 + 1) % 4
            fib = (b4 + 3) % 4

            @pl.when(j + 1 < NCH)
            def _advance():
                @pl.when(j >= 1)
                def _drain_prev():
                    wait_scatter(fib, nrb)

                @pl.when(j + 3 < NCH)
                def _prefetch_idx():
                    issue_idx(j + 3, fib)

                wait_idx(j + 1, nib)
                issue_gather(j + 1, nib, nrb)

            wait_gather(ib, rb)
            issue_scatter(ib, rb)

    wait_scatter(2, 0)
    wait_scatter(3, 1)
    plsc.subcore_barrier()
    pltpu.sync_copy(acc.at[pl.ds(sid * DRAIN, DRAIN)],
                    out_hbm.at[cid].at[pl.ds(sid * DRAIN, DRAIN)])


def _k2_body(x_ref, degp_ref, u1_ref, dis_ref):
    j = pl.program_id(0)
    p = degp_ref[...]
    deg = p[0, :, 0:1] + p[1, :, 0:1] + 1.0
    dis = lax.rsqrt(deg)
    rows = lax.broadcasted_iota(jnp.int32, (BLK, 1), 0) + j * BLK
    mask = rows < N
    u1_ref[...] = jnp.where(mask, x_ref[...] * dis, 0.0)
    dis_ref[...] = dis


_tc_k2 = pl.pallas_call(
    _k2_body,
    grid=(GRID,),
    in_specs=[
        pl.BlockSpec((BLK, 128), lambda j: (j, 0)),
        pl.BlockSpec((NC, BLK, DEGW), lambda j: (0, j, 0)),
    ],
    out_specs=[
        pl.BlockSpec((BLK, 128), lambda j: (j, 0)),
        pl.BlockSpec((BLK, 1), lambda j: (j, 0)),
    ],
    out_shape=[
        jax.ShapeDtypeStruct((NPAD, 128), jnp.float32),
        jax.ShapeDtypeStruct((NPAD, 1), jnp.float32),
    ],
)


def _k4_body(s_ref, u1_ref, dis_ref, w_ref, b_ref, u2_ref):
    j = pl.program_id(0)
    s = s_ref[0] + s_ref[1] + u1_ref[...]
    ax = dis_ref[...] * s
    h = jnp.dot(ax, w_ref[...], preferred_element_type=jnp.float32)
    h = jnp.maximum(h + b_ref[...], 0.0)
    rows = lax.broadcasted_iota(jnp.int32, (BLK, 1), 0) + j * BLK
    mask = rows < N
    u2_ref[...] = jnp.where(mask, dis_ref[...] * h, 0.0)


_tc_k4 = pl.pallas_call(
    _k4_body,
    grid=(GRID,),
    in_specs=[
        pl.BlockSpec((NC, BLK, 128), lambda j: (0, j, 0)),
        pl.BlockSpec((BLK, 128), lambda j: (j, 0)),
        pl.BlockSpec((BLK, 1), lambda j: (j, 0)),
        pl.BlockSpec((128, 128), lambda j: (0, 0)),
        pl.BlockSpec((1, 128), lambda j: (0, 0)),
    ],
    out_specs=pl.BlockSpec((BLK, 128), lambda j: (j, 0)),
    out_shape=jax.ShapeDtypeStruct((NPAD, 128), jnp.float32),
)


def _k6_body(s_ref, u2_ref, dis_ref, wmu_ref, bmu_ref, wls_ref, bls_ref,
             mu_ref, ls_ref):
    s = s_ref[0] + s_ref[1] + u2_ref[...]
    ah = dis_ref[...] * s
    mu_ref[...] = (
        jnp.dot(ah, wmu_ref[...], preferred_element_type=jnp.float32)
        + bmu_ref[...])
    ls_ref[...] = (
        jnp.dot(ah, wls_ref[...], preferred_element_type=jnp.float32)
        + bls_ref[...])


_tc_k6 = pl.pallas_call(
    _k6_body,
    grid=(GRID,),
    in_specs=[
        pl.BlockSpec((NC, BLK, 128), lambda j: (0, j, 0)),
        pl.BlockSpec((BLK, 128), lambda j: (j, 0)),
        pl.BlockSpec((BLK, 1), lambda j: (j, 0)),
        pl.BlockSpec((128, 128), lambda j: (0, 0)),
        pl.BlockSpec((1, 128), lambda j: (0, 0)),
        pl.BlockSpec((128, 128), lambda j: (0, 0)),
        pl.BlockSpec((1, 128), lambda j: (0, 0)),
    ],
    out_specs=[
        pl.BlockSpec((BLK, 128), lambda j: (j, 0)),
        pl.BlockSpec((BLK, 128), lambda j: (j, 0)),
    ],
    out_shape=[
        jax.ShapeDtypeStruct((N, 128), jnp.float32),
        jax.ShapeDtypeStruct((N, 128), jnp.float32),
    ],
)


def kernel(x, edge_index, W1, b1, Wmu, bmu, Wls, bls):
    ei = edge_index.astype(jnp.int32)
    npads = EPAD - E
    pad_idx = N + (jnp.arange(npads, dtype=jnp.int32) % (NPAD - N))
    srcp = jnp.concatenate([ei[0], pad_idx]).reshape(EPAD // C, 1, C)
    dstp = jnp.concatenate([ei[1], pad_idx]).reshape(EPAD // C, 1, C)
    idx = jnp.concatenate([srcp, dstp], axis=1)

    ones = jnp.ones((C, DEGW), jnp.float32)
    zeros128 = jnp.zeros((DRAIN, 128), jnp.float32)

    degp = _sc_deg(idx, zeros128, ones)
    u1, dis = _tc_k2(x, degp)
    s1 = _sc_apply(u1, idx, zeros128)
    u2 = _tc_k4(s1, u1, dis, W1, b1.reshape(1, 128))
    s2 = _sc_apply(u2, idx, zeros128)
    mu, ls = _tc_k6(s2, u2, dis, Wmu, bmu.reshape(1, 128),
                    Wls, bls.reshape(1, 128))
    return (mu, ls)
